# baseline (speedup 1.0000x reference)
import jax
import jax.numpy as jnp
from jax.experimental import pallas as pl

_LAYERS = 3
_K = 1024
_D = 64
_BLK = 1024


def _rowsum64(s):
    acc = s[:, 0:8]
    for k in range(1, s.shape[1] // 8):
        acc = acc + s[:, 8 * k:8 * k + 8]
    a = acc[:, :4] + acc[:, 4:8]
    a = a[:, :2] + a[:, 2:4]
    return a[:, 0:1] + a[:, 1:2]


def _rvq_body(z_ref, cb_ref, q_ref, idx_ref):
    residual = z_ref[...]
    b = residual.shape[0]
    iota_f = jax.lax.broadcasted_iota(jnp.int32, (b, _K), 1).astype(jnp.float32)
    quant = jnp.zeros_like(residual)
    dn = (((1,), (0,)), ((), ()))
    for l in range(_LAYERS):
        cb = cb_ref[l]
        r2 = _rowsum64(residual * residual)
        c2 = jnp.sum(cb * cb, axis=1)[None, :]
        dot = jax.lax.dot_general(residual, cb, (((1,), (1,)), ((), ())),
                                  preferred_element_type=jnp.float32)
        d2 = r2 + c2 - 2.0 * dot
        dist = jnp.sqrt(jnp.maximum(d2, 0.0))
        m = jnp.min(dist, axis=1, keepdims=True)
        idx_f = jnp.min(jnp.where(dist == m, iota_f, 2048.0),
                        axis=1, keepdims=True)
        onehot = (iota_f == idx_f).astype(jnp.float32)
        cw = jax.lax.dot_general(onehot, cb, dn, preferred_element_type=jnp.float32,
                                 precision=jax.lax.Precision.HIGHEST)
        residual = residual - cw
        quant = quant + cw
        idx_ref[l, :] = idx_f[:, 0].astype(jnp.int32)
    q_ref[...] = quant


def kernel(z, codebooks):
    n, d = z.shape
    grid = (n // _BLK,)
    q, idx = pl.pallas_call(
        _rvq_body,
        grid=grid,
        in_specs=[
            pl.BlockSpec((_BLK, d), lambda i: (i, 0)),
            pl.BlockSpec((_LAYERS, _K, d), lambda i: (0, 0, 0)),
        ],
        out_specs=[
            pl.BlockSpec((_BLK, d), lambda i: (i, 0)),
            pl.BlockSpec((_LAYERS, _BLK), lambda i: (0, i)),
        ],
        out_shape=[
            jax.ShapeDtypeStruct((n, d), jnp.float32),
            jax.ShapeDtypeStruct((_LAYERS, n), jnp.int32),
        ],
    )(z, codebooks)
    return (q, idx)


# speedup vs baseline: 1.9773x; 1.9773x over previous
import jax
import jax.numpy as jnp
from jax.experimental import pallas as pl

_LAYERS = 3
_K = 1024
_D = 64
_BLK = 1024


def _colsum64(s):
    acc = s[0:8, :]
    for k in range(1, 8):
        acc = acc + s[8 * k:8 * k + 8, :]
    a = acc[0:4, :] + acc[4:8, :]
    a = a[0:2, :] + a[2:4, :]
    return a[0:1, :] + a[1:2, :]


def _rvq_body(zT_ref, cb_ref, qT_ref, idx_ref):
    rT = zT_ref[...]
    b = rT.shape[1]
    iota_f = jax.lax.broadcasted_iota(jnp.int32, (_K, b), 0).astype(jnp.float32)
    quant = jnp.zeros_like(rT)
    for l in range(_LAYERS):
        cb = cb_ref[l]
        r2 = _colsum64(rT * rT)
        c2 = jnp.sum(cb * cb, axis=1)[:, None]
        dotT = jax.lax.dot_general(cb, rT, (((1,), (0,)), ((), ())),
                                   preferred_element_type=jnp.float32)
        d2 = r2 + c2 - 2.0 * dotT
        dist = jnp.sqrt(jnp.maximum(d2, 0.0))
        m = jnp.min(dist, axis=0, keepdims=True)
        idx_f = jnp.min(jnp.where(dist == m, iota_f, 2048.0),
                        axis=0, keepdims=True)
        onehot = (iota_f == idx_f).astype(jnp.float32)
        cwT = jax.lax.dot_general(cb, onehot, (((0,), (0,)), ((), ())),
                                  preferred_element_type=jnp.float32,
                                  precision=jax.lax.Precision.HIGHEST)
        rT = rT - cwT
        quant = quant + cwT
        idx_ref[l, :] = idx_f[0, :].astype(jnp.int32)
    qT_ref[...] = quant


def kernel(z, codebooks):
    n, d = z.shape
    zT = z.T
    grid = (n // _BLK,)
    qT, idx = pl.pallas_call(
        _rvq_body,
        grid=grid,
        in_specs=[
            pl.BlockSpec((d, _BLK), lambda i: (0, i)),
            pl.BlockSpec((_LAYERS, _K, d), lambda i: (0, 0, 0)),
        ],
        out_specs=[
            pl.BlockSpec((d, _BLK), lambda i: (0, i)),
            pl.BlockSpec((_LAYERS, _BLK), lambda i: (0, i)),
        ],
        out_shape=[
            jax.ShapeDtypeStruct((d, n), jnp.float32),
            jax.ShapeDtypeStruct((_LAYERS, n), jnp.int32),
        ],
    )(zT, codebooks)
    return (qT.T, idx)


# speedup vs baseline: 2.7933x; 1.4127x over previous
import jax
import jax.numpy as jnp
from jax.experimental import pallas as pl

_LAYERS = 3
_K = 1024
_D = 64
_BLK = 1024


def _colsum64(s):
    acc = s[0:8, :]
    for k in range(1, 8):
        acc = acc + s[8 * k:8 * k + 8, :]
    a = acc[0:4, :] + acc[4:8, :]
    a = a[0:2, :] + a[2:4, :]
    return a[0:1, :] + a[1:2, :]


def _rvq_body(zT_ref, cb_ref, qT_ref, idx_ref):
    rT = zT_ref[...]
    b = rT.shape[1]
    iota_f = jax.lax.broadcasted_iota(jnp.int32, (_K, b), 0).astype(jnp.float32)
    quant = jnp.zeros_like(rT)
    for l in range(_LAYERS):
        cb = cb_ref[l]
        r2 = _colsum64(rT * rT)
        c2 = jnp.sum(cb * cb, axis=1)[:, None]
        dotT = jax.lax.dot_general(cb, rT, (((1,), (0,)), ((), ())),
                                   preferred_element_type=jnp.float32)
        d2 = r2 + c2 - 2.0 * dotT
        dist = jnp.sqrt(jnp.maximum(d2, 0.0))
        m = jnp.min(dist, axis=0, keepdims=True)
        idx_f = jnp.min(jnp.where(dist == m, iota_f, 2048.0),
                        axis=0, keepdims=True)
        onehot = (iota_f == idx_f).astype(jnp.bfloat16)
        hi = cb.astype(jnp.bfloat16)
        rem = cb - hi.astype(jnp.float32)
        mid = rem.astype(jnp.bfloat16)
        lo = (rem - mid.astype(jnp.float32)).astype(jnp.bfloat16)
        dn = (((0,), (0,)), ((), ()))
        cwT = ((jax.lax.dot_general(hi, onehot, dn, preferred_element_type=jnp.float32)
                + jax.lax.dot_general(mid, onehot, dn, preferred_element_type=jnp.float32))
               + jax.lax.dot_general(lo, onehot, dn, preferred_element_type=jnp.float32))
        rT = rT - cwT
        quant = quant + cwT
        idx_ref[l, :] = idx_f[0, :].astype(jnp.int32)
    qT_ref[...] = quant


def kernel(z, codebooks):
    n, d = z.shape
    zT = z.T
    grid = (n // _BLK,)
    qT, idx = pl.pallas_call(
        _rvq_body,
        grid=grid,
        in_specs=[
            pl.BlockSpec((d, _BLK), lambda i: (0, i)),
            pl.BlockSpec((_LAYERS, _K, d), lambda i: (0, 0, 0)),
        ],
        out_specs=[
            pl.BlockSpec((d, _BLK), lambda i: (0, i)),
            pl.BlockSpec((_LAYERS, _BLK), lambda i: (0, i)),
        ],
        out_shape=[
            jax.ShapeDtypeStruct((d, n), jnp.float32),
            jax.ShapeDtypeStruct((_LAYERS, n), jnp.int32),
        ],
    )(zT, codebooks)
    return (qT.T, idx)
